# Initial kernel scaffold; baseline (speedup 1.0000x reference)
import jax
import jax.numpy as jnp
from jax.experimental import pallas as pl


def kernel(x, edge_index, edge_attr, W1, b1, gamma1, beta1, W2, b2, gamma2, beta2):
    raise NotImplementedError("write your pallas kernel here")


# speedup vs baseline: 15.7028x; 15.7028x over previous
import functools

import jax
import jax.numpy as jnp
from jax import lax
from jax.experimental import pallas as pl
from jax.experimental.pallas import tpu as pltpu
from jax.experimental.pallas import tpu_sc as plsc

N = 10000
E = 320000
D = 128
EPSILON = 1e-5

NC = 2
NS = 16
NW = NC * NS
EPW = E // NW
B = 80
NCHUNK = EPW // B
NPAD = 10240
RPS = NPAD // NS

_mesh = plsc.VectorSubcoreMesh(core_axis_name="c", subcore_axis_name="s")
_sc_params = pltpu.CompilerParams(needs_layout_passes=False,
                                 use_tc_tiling_on_sc=False)


@functools.partial(
    pl.kernel,
    out_type=jax.ShapeDtypeStruct((NW, NPAD), jnp.float32),
    mesh=_mesh,
    compiler_params=_sc_params,
    scratch_types=[
        pltpu.VMEM((EPW,), jnp.int32),
        pltpu.VMEM((EPW,), jnp.float32),
        pltpu.VMEM((NPAD,), jnp.float32),
    ],
)
def _deg_partials(dst_hbm, w_hbm, out_hbm, dstv, wv, degv):
    wid = lax.axis_index("s") * NC + lax.axis_index("c")
    pltpu.sync_copy(dst_hbm.at[wid], dstv)
    pltpu.sync_copy(w_hbm.at[wid], wv)

    zero = jnp.zeros((16,), jnp.float32)

    def _zero(i, carry):
        degv[pl.ds(i * 16, 16)] = zero
        return carry

    lax.fori_loop(0, NPAD // 16, _zero, 0)

    def _scat(i, carry):
        idx = dstv[pl.ds(i * 16, 16)]
        ww = wv[pl.ds(i * 16, 16)]
        plsc.addupdate_scatter(degv, [idx], ww)
        return carry

    lax.fori_loop(0, EPW // 16, _scat, 0)
    pltpu.sync_copy(degv, out_hbm.at[wid])


@functools.partial(
    pl.kernel,
    out_type=jax.ShapeDtypeStruct((NC, NPAD, D), jnp.float32),
    mesh=_mesh,
    compiler_params=_sc_params,
    scratch_types=[
        pltpu.VMEM((NCHUNK, B), jnp.int32),
        pltpu.VMEM((NCHUNK, B), jnp.int32),
        pltpu.VMEM((NCHUNK, B), jnp.float32),
        pltpu.VMEM((B, D), jnp.float32),
        pltpu.VMEM_SHARED((NPAD, D), jnp.float32),
        pltpu.SemaphoreType.DMA,
    ],
)
def _message_pass(y_hbm, src_hbm, dst_hbm, w_hbm, z_hbm, out_hbm,
                  srcv, dstv, wv, rows, accsh, sem):
    cid = lax.axis_index("c")
    sid = lax.axis_index("s")
    wid = sid * NC + cid

    pltpu.sync_copy(src_hbm.at[wid], srcv)
    pltpu.sync_copy(dst_hbm.at[wid], dstv)
    pltpu.sync_copy(w_hbm.at[wid], wv)

    r0 = sid * RPS
    pltpu.sync_copy(z_hbm, accsh.at[pl.ds(r0, RPS)])
    plsc.subcore_barrier()

    def _chunk(j, carry):
        pltpu.async_copy(y_hbm.at[srcv.at[j]], rows, sem).wait()

        def _grp(gi, c2):
            wvec = wv[j, pl.ds(gi * 16, 16)]
            for lane in range(16):
                ws = wvec[lane]
                e = gi * 16 + lane
                for g in range(D // 16):
                    sl = pl.ds(g * 16, 16)
                    rows[e, sl] = rows[e, sl] * ws
            return c2

        lax.fori_loop(0, B // 16, _grp, 0)
        pltpu.sync_copy(rows, accsh.at[dstv.at[j]], add=True)
        return carry

    lax.fori_loop(0, NCHUNK, _chunk, 0)
    plsc.subcore_barrier()
    pltpu.sync_copy(accsh.at[pl.ds(r0, RPS)], out_hbm.at[cid, pl.ds(r0, RPS)])


def _dinv_body(p_ref, o_ref):
    deg = jnp.sum(p_ref[...], axis=0) + 1.0
    o_ref[...] = jnp.where(deg > 0, lax.rsqrt(deg), 0.0)


def _mm_scale_body(x_ref, w_ref, dinv_ref, o_ref):
    o_ref[...] = (
        jnp.dot(x_ref[...], w_ref[...], preferred_element_type=jnp.float32)
        * dinv_ref[...]
    )


def _combine_stats_body(a0_ref, a1_ref, y_ref, dinv_ref, b_ref, h_ref, st_ref):
    i = pl.program_id(0)
    h = (a0_ref[...] + a1_ref[...] + y_ref[...]) * dinv_ref[...] + b_ref[...]
    h_ref[...] = h
    s1 = jnp.sum(h, axis=0, keepdims=True)
    s2 = jnp.sum(h * h, axis=0, keepdims=True)
    upd = jnp.concatenate([s1, s2, jnp.zeros((6, D), jnp.float32)], axis=0)

    @pl.when(i == 0)
    def _():
        st_ref[...] = upd

    @pl.when(i > 0)
    def _():
        st_ref[...] = st_ref[...] + upd


def _bn_scale_shift(st_ref, g_ref, be_ref):
    mean = st_ref[0:1, :] * (1.0 / N)
    ex2 = st_ref[1:2, :] * (1.0 / N)
    var = ex2 - mean * mean
    a = g_ref[...] * lax.rsqrt(var + EPSILON)
    s = be_ref[...] - mean * a
    return a, s


def _bn_mm_scale_body(h_ref, w_ref, st_ref, g_ref, be_ref, dinv_ref, o_ref):
    a, s = _bn_scale_shift(st_ref, g_ref, be_ref)
    hb = h_ref[...] * a + s
    o_ref[...] = (
        jnp.dot(hb, w_ref[...], preferred_element_type=jnp.float32)
        * dinv_ref[...]
    )


def _bn_apply_body(h_ref, st_ref, g_ref, be_ref, o_ref):
    a, s = _bn_scale_shift(st_ref, g_ref, be_ref)
    o_ref[...] = h_ref[...] * a + s


RB = 1000
GRID = N // RB

_row_spec = pl.BlockSpec((RB, D), lambda i: (i, 0))
_dinv_spec = pl.BlockSpec((RB, 1), lambda i: (i, 0))
_full_spec = pl.BlockSpec((D, D), lambda i: (0, 0))
_vec_spec = pl.BlockSpec((1, D), lambda i: (0, 0))
_st_spec = pl.BlockSpec((8, D), lambda i: (0, 0))

_f32 = jnp.float32


def _dinv_call(parts):
    return pl.pallas_call(
        _dinv_body,
        out_shape=jax.ShapeDtypeStruct((NPAD // D, D), _f32),
    )(parts)


def _mm_scale_call(x, w, dinv2d):
    return pl.pallas_call(
        _mm_scale_body,
        grid=(GRID,),
        in_specs=[_row_spec, _full_spec, _dinv_spec],
        out_specs=_row_spec,
        out_shape=jax.ShapeDtypeStruct((N, D), _f32),
    )(x, w, dinv2d)


def _combine_stats_call(a0, a1, y, dinv2d, b2d):
    return pl.pallas_call(
        _combine_stats_body,
        grid=(GRID,),
        in_specs=[_row_spec, _row_spec, _row_spec, _dinv_spec, _vec_spec],
        out_specs=[_row_spec, _st_spec],
        out_shape=[
            jax.ShapeDtypeStruct((N, D), _f32),
            jax.ShapeDtypeStruct((8, D), _f32),
        ],
    )(a0, a1, y, dinv2d, b2d)


def _bn_mm_scale_call(h, w, st, g2d, be2d, dinv2d):
    return pl.pallas_call(
        _bn_mm_scale_body,
        grid=(GRID,),
        in_specs=[_row_spec, _full_spec, _st_spec, _vec_spec, _vec_spec,
                  _dinv_spec],
        out_specs=_row_spec,
        out_shape=jax.ShapeDtypeStruct((N, D), _f32),
    )(h, w, st, g2d, be2d, dinv2d)


def _bn_apply_call(h, st, g2d, be2d):
    return pl.pallas_call(
        _bn_apply_body,
        grid=(GRID,),
        in_specs=[_row_spec, _st_spec, _vec_spec, _vec_spec],
        out_specs=_row_spec,
        out_shape=jax.ShapeDtypeStruct((N, D), _f32),
    )(h, st, g2d, be2d)


def kernel(x, edge_index, edge_attr, W1, b1, gamma1, beta1,
           W2, b2, gamma2, beta2):
    src = edge_index[0].astype(jnp.int32)
    dst = edge_index[1].astype(jnp.int32)
    w = edge_attr.astype(jnp.float32)

    dstf = dst.reshape(NW, EPW)
    wf = w.reshape(NW, EPW)
    src3 = src.reshape(NW, NCHUNK, B)
    dst3 = dst.reshape(NW, NCHUNK, B)
    w3 = w.reshape(NW, NCHUNK, B)
    zrows = jnp.zeros((RPS, D), _f32)

    b1_2d = b1.reshape(1, D)
    b2_2d = b2.reshape(1, D)
    g1_2d = gamma1.reshape(1, D)
    g2_2d = gamma2.reshape(1, D)
    be1_2d = beta1.reshape(1, D)
    be2_2d = beta2.reshape(1, D)

    parts = _deg_partials(dstf, wf).reshape(NW, NPAD // D, D)
    dinv2d = _dinv_call(parts).reshape(NPAD)[:N].reshape(N, 1)

    y1 = _mm_scale_call(x, W1, dinv2d)
    acc1 = _message_pass(y1, src3, dst3, w3, zrows)
    h1, st1 = _combine_stats_call(acc1[0], acc1[1], y1, dinv2d, b1_2d)

    y2 = _bn_mm_scale_call(h1, W2, st1, g1_2d, be1_2d, dinv2d)
    acc2 = _message_pass(y2, src3, dst3, w3, zrows)
    h2, st2 = _combine_stats_call(acc2[0], acc2[1], y2, dinv2d, b2_2d)

    return _bn_apply_call(h2, st2, g2_2d, be2_2d)
